# baseline (speedup 1.0000x reference)
import functools

import jax
import jax.numpy as jnp
import numpy as np
from jax.experimental import pallas as pl
from jax.experimental.pallas import tpu as pltpu
from jax.experimental.pallas import tpu_sc as plsc

B = 16384
NUM_NUMERIC = 13
N_FIELDS = 26
CARD = 100000
EMB_DIM = 32
EMB_WIDTH = N_FIELDS * EMB_DIM
H1, H2 = 64, 32
EPS = 1e-5
GW = 128
BLK = 2048

CHUNK = 4096
NBLK = -(-CARD // CHUNK)
FG = 4
NG = -(-N_FIELDS // FG)
PLANE = NBLK * CHUNK
CHUNKS = ((0, 1), (1, 2), (3, 2), (5, 2))
CHUNK_NF = tuple(min(N_FIELDS - g0 * FG, ng * FG) for g0, ng in CHUNKS)


def _detranspose(tab_T, g0, ng):
    def body(in_ref, out_ref):
        x = in_ref[...]
        out_ref[0] = x.reshape(FG * EMB_DIM, CHUNK).T

    return pl.pallas_call(
        body,
        grid=(ng, NBLK),
        in_specs=[pl.BlockSpec((FG, EMB_DIM, CHUNK), lambda g, c: (g0 + g, 0, c))],
        out_specs=pl.BlockSpec((1, CHUNK, FG * EMB_DIM), lambda g, c: (g, c, 0)),
        out_shape=jax.ShapeDtypeStruct((ng, PLANE, FG * EMB_DIM), jnp.float32),
        compiler_params=pltpu.CompilerParams(
            dimension_semantics=("parallel", "parallel")
        ),
    )(tab_T)


def _sc_gather(tables_lin, flat_idx, nidx):
    mesh = plsc.VectorSubcoreMesh(core_axis_name="core", subcore_axis_name="subcore")

    @functools.partial(
        pl.kernel,
        out_type=jax.ShapeDtypeStruct((nidx, EMB_DIM), jnp.float32),
        mesh=mesh,
        compiler_params=pltpu.CompilerParams(use_tc_tiling_on_sc=False),
    )
    def gather_kernel(tab_hbm, idx_hbm, out_hbm):
        def body(idx_v, out_v):
            pltpu.sync_copy(tab_hbm.at[idx_v.at[0]], out_v)

        pltpu.emit_pipeline(
            body,
            grid=(nidx // GW,),
            in_specs=[pl.BlockSpec((1, GW), index_map=lambda i: (0, i))],
            out_specs=[pl.BlockSpec((GW, EMB_DIM), index_map=lambda i: (i, 0))],
            core_axis_name=("core", "subcore"),
            dimension_semantics=(pltpu.PARALLEL,),
        )(idx_hbm, out_hbm)

    return gather_kernel(tables_lin, flat_idx)


def _mlp_body(xn, em0, em1, em2, em3, w1n, w10, w11, w12, w13,
              b1r, g1r, be1r, w2, b2r, g2r, be2r, w3, b3r, out):
    s = np.float32(1.0 / np.sqrt(1.0 + EPS))
    h = jnp.dot(em0[...], w10[...], preferred_element_type=jnp.float32)
    h = h + jnp.dot(em1[...], w11[...], preferred_element_type=jnp.float32)
    h = h + jnp.dot(em2[...], w12[...], preferred_element_type=jnp.float32)
    h = h + jnp.dot(em3[...], w13[...], preferred_element_type=jnp.float32)
    h = h + jnp.dot(xn[...], w1n[...], preferred_element_type=jnp.float32)
    a1 = g1r[...] * s
    h = h * a1 + (b1r[...] * a1 + be1r[...])
    h = jnp.maximum(h, 0.0)
    h2 = jnp.dot(h, w2[...], preferred_element_type=jnp.float32)
    a2 = g2r[...] * s
    h2 = h2 * a2 + (b2r[...] * a2 + be2r[...])
    h2 = jnp.maximum(h2, 0.0)
    out[...] = jnp.dot(h2, w3[...], preferred_element_type=jnp.float32) + b3r[...]


def _mlp(x_num, embs, W1n, W1s, b1, g1, be1, W2, b2, g2, be2, W3, b3):
    grid = (B // BLK,)
    row_spec = lambda w: pl.BlockSpec((BLK, w), lambda i: (i, 0))
    full_spec = lambda a: pl.BlockSpec(a.shape, lambda i: (0, 0))
    args = (x_num, *embs, W1n, *W1s, b1, g1, be1, W2, b2, g2, be2, W3, b3)
    in_specs = [row_spec(NUM_NUMERIC)] + [
        row_spec(nf * EMB_DIM) for nf in CHUNK_NF
    ] + [full_spec(a) for a in args[5:]]
    return pl.pallas_call(
        _mlp_body,
        grid=grid,
        in_specs=in_specs,
        out_specs=pl.BlockSpec((BLK, 1), lambda i: (i, 0)),
        out_shape=jax.ShapeDtypeStruct((B, 1), jnp.float32),
        compiler_params=pltpu.CompilerParams(dimension_semantics=("parallel",)),
    )(*args)


def _chunk_idx(x_cat_chunk, nf):
    f = jnp.arange(nf, dtype=jnp.int32)
    offs = (4 * PLANE * (f // FG) + f % FG)[None, :]
    return (x_cat_chunk * 4 + offs).reshape(1, B * nf)


def kernel(x_num, x_cat, emb_tables, W1, b1, g1, be1, W2, b2, g2, be2, W3, b3):
    tab_T = jnp.swapaxes(emb_tables, 1, 2)
    embs, W1s = [], []
    f0 = 0
    for (g0, ng), nf in zip(CHUNKS, CHUNK_NF):
        lin = _detranspose(tab_T, g0, ng).reshape(ng * PLANE * FG, EMB_DIM)
        idx = _chunk_idx(x_cat[:, f0:f0 + nf], nf)
        embs.append(_sc_gather(lin, idx, B * nf).reshape(B, nf * EMB_DIM))
        W1s.append(W1[NUM_NUMERIC + f0 * EMB_DIM:NUM_NUMERIC + (f0 + nf) * EMB_DIM])
        f0 += nf
    W1n = W1[:NUM_NUMERIC]
    vec = lambda v: v.reshape(1, -1)
    return _mlp(x_num, embs, W1n, W1s, vec(b1), vec(g1), vec(be1),
                W2, vec(b2), vec(g2), vec(be2), W3, vec(b3))


# speedup vs baseline: 1.0690x; 1.0690x over previous
import functools

import jax
import jax.numpy as jnp
import numpy as np
from jax.experimental import pallas as pl
from jax.experimental.pallas import tpu as pltpu
from jax.experimental.pallas import tpu_sc as plsc

B = 16384
NUM_NUMERIC = 13
N_FIELDS = 26
CARD = 100000
EMB_DIM = 32
EMB_WIDTH = N_FIELDS * EMB_DIM
H1, H2 = 64, 32
EPS = 1e-5
GW = 128
BLK = 2048

CHUNK = 4096
NBLK = -(-CARD // CHUNK)
FG = 4
NG = -(-N_FIELDS // FG)
PLANE = NBLK * CHUNK
CHUNKS = ((0, 4), (4, 3))
CHUNK_NF = tuple(min(N_FIELDS - g0 * FG, ng * FG) for g0, ng in CHUNKS)


def _detranspose(tab_T, g0, ng):
    def body(in_ref, out_ref):
        x = in_ref[...]
        out_ref[0] = x.reshape(FG * EMB_DIM, CHUNK).T

    return pl.pallas_call(
        body,
        grid=(ng, NBLK),
        in_specs=[pl.BlockSpec((FG, EMB_DIM, CHUNK), lambda g, c: (g0 + g, 0, c))],
        out_specs=pl.BlockSpec((1, CHUNK, FG * EMB_DIM), lambda g, c: (g, c, 0)),
        out_shape=jax.ShapeDtypeStruct((ng, PLANE, FG * EMB_DIM), jnp.float32),
        compiler_params=pltpu.CompilerParams(
            dimension_semantics=("parallel", "parallel")
        ),
    )(tab_T)


def _sc_gather(tables_lin, flat_idx, nidx):
    mesh = plsc.VectorSubcoreMesh(core_axis_name="core", subcore_axis_name="subcore")

    @functools.partial(
        pl.kernel,
        out_type=jax.ShapeDtypeStruct((nidx, EMB_DIM), jnp.float32),
        mesh=mesh,
        compiler_params=pltpu.CompilerParams(use_tc_tiling_on_sc=False),
    )
    def gather_kernel(tab_hbm, idx_hbm, out_hbm):
        def body(idx_v, out_v):
            pltpu.sync_copy(tab_hbm.at[idx_v.at[0]], out_v)

        pltpu.emit_pipeline(
            body,
            grid=(nidx // GW,),
            in_specs=[pl.BlockSpec((1, GW), index_map=lambda i: (0, i))],
            out_specs=[pl.BlockSpec((GW, EMB_DIM), index_map=lambda i: (i, 0))],
            core_axis_name=("core", "subcore"),
            dimension_semantics=(pltpu.PARALLEL,),
        )(idx_hbm, out_hbm)

    return gather_kernel(tables_lin, flat_idx)


def _mlp_body(*refs):
    n = len(CHUNKS)
    xn, ems = refs[0], refs[1:1 + n]
    w1n, w1s = refs[1 + n], refs[2 + n:2 + 2 * n]
    b1r, g1r, be1r, w2, b2r, g2r, be2r, w3, b3r, out = refs[2 + 2 * n:]
    s = np.float32(1.0 / np.sqrt(1.0 + EPS))
    h = jnp.dot(xn[...], w1n[...], preferred_element_type=jnp.float32)
    for em, w1e in zip(ems, w1s):
        h = h + jnp.dot(em[...], w1e[...], preferred_element_type=jnp.float32)
    a1 = g1r[...] * s
    h = h * a1 + (b1r[...] * a1 + be1r[...])
    h = jnp.maximum(h, 0.0)
    h2 = jnp.dot(h, w2[...], preferred_element_type=jnp.float32)
    a2 = g2r[...] * s
    h2 = h2 * a2 + (b2r[...] * a2 + be2r[...])
    h2 = jnp.maximum(h2, 0.0)
    out[...] = jnp.dot(h2, w3[...], preferred_element_type=jnp.float32) + b3r[...]


def _mlp(x_num, embs, W1n, W1s, b1, g1, be1, W2, b2, g2, be2, W3, b3):
    grid = (B // BLK,)
    row_spec = lambda w: pl.BlockSpec((BLK, w), lambda i: (i, 0))
    full_spec = lambda a: pl.BlockSpec(a.shape, lambda i: (0, 0))
    args = (x_num, *embs, W1n, *W1s, b1, g1, be1, W2, b2, g2, be2, W3, b3)
    in_specs = [row_spec(NUM_NUMERIC)] + [
        row_spec(nf * EMB_DIM) for nf in CHUNK_NF
    ] + [full_spec(a) for a in args[1 + len(CHUNK_NF):]]
    return pl.pallas_call(
        _mlp_body,
        grid=grid,
        in_specs=in_specs,
        out_specs=pl.BlockSpec((BLK, 1), lambda i: (i, 0)),
        out_shape=jax.ShapeDtypeStruct((B, 1), jnp.float32),
        compiler_params=pltpu.CompilerParams(dimension_semantics=("parallel",)),
    )(*args)


def _chunk_idx(x_cat_chunk, nf):
    f = jnp.arange(nf, dtype=jnp.int32)
    offs = (4 * PLANE * (f // FG) + f % FG)[None, :]
    return (x_cat_chunk * 4 + offs).reshape(1, B * nf)


def kernel(x_num, x_cat, emb_tables, W1, b1, g1, be1, W2, b2, g2, be2, W3, b3):
    tab_T = jnp.swapaxes(emb_tables, 1, 2)
    embs, W1s = [], []
    f0 = 0
    for (g0, ng), nf in zip(CHUNKS, CHUNK_NF):
        lin = _detranspose(tab_T, g0, ng).reshape(ng * PLANE * FG, EMB_DIM)
        idx = _chunk_idx(x_cat[:, f0:f0 + nf], nf)
        embs.append(_sc_gather(lin, idx, B * nf).reshape(B, nf * EMB_DIM))
        W1s.append(W1[NUM_NUMERIC + f0 * EMB_DIM:NUM_NUMERIC + (f0 + nf) * EMB_DIM])
        f0 += nf
    W1n = W1[:NUM_NUMERIC]
    vec = lambda v: v.reshape(1, -1)
    return _mlp(x_num, embs, W1n, W1s, vec(b1), vec(g1), vec(be1),
                W2, vec(b2), vec(g2), vec(be2), W3, vec(b3))


# speedup vs baseline: 1.1527x; 1.0783x over previous
import functools

import jax
import jax.numpy as jnp
import numpy as np
from jax.experimental import pallas as pl
from jax.experimental.pallas import tpu as pltpu
from jax.experimental.pallas import tpu_sc as plsc

B = 16384
NUM_NUMERIC = 13
N_FIELDS = 26
CARD = 100000
EMB_DIM = 32
EMB_WIDTH = N_FIELDS * EMB_DIM
H1, H2 = 64, 32
EPS = 1e-5
GW = 128
BLK = 2048

CHUNK = 8192
NBLK = -(-CARD // CHUNK)
FG = 4
NG = -(-N_FIELDS // FG)
PLANE = NBLK * CHUNK
CHUNKS = ((0, 4), (4, 3))
CHUNK_NF = tuple(min(N_FIELDS - g0 * FG, ng * FG) for g0, ng in CHUNKS)


def _detranspose(tab_T, g0, ng):
    def body(in_ref, out_ref):
        x = in_ref[...]
        out_ref[0] = x.reshape(FG * EMB_DIM, CHUNK).T

    return pl.pallas_call(
        body,
        grid=(ng, NBLK),
        in_specs=[pl.BlockSpec((FG, EMB_DIM, CHUNK), lambda g, c: (g0 + g, 0, c))],
        out_specs=pl.BlockSpec((1, CHUNK, FG * EMB_DIM), lambda g, c: (g, c, 0)),
        out_shape=jax.ShapeDtypeStruct((ng, PLANE, FG * EMB_DIM), jnp.float32),
        compiler_params=pltpu.CompilerParams(
            dimension_semantics=("parallel", "parallel")
        ),
    )(tab_T)


def _sc_gather(tables_lin, flat_idx, nidx):
    mesh = plsc.VectorSubcoreMesh(core_axis_name="core", subcore_axis_name="subcore")

    @functools.partial(
        pl.kernel,
        out_type=jax.ShapeDtypeStruct((nidx, EMB_DIM), jnp.float32),
        mesh=mesh,
        compiler_params=pltpu.CompilerParams(use_tc_tiling_on_sc=False),
    )
    def gather_kernel(tab_hbm, idx_hbm, out_hbm):
        def body(idx_v, out_v):
            pltpu.sync_copy(tab_hbm.at[idx_v.at[0]], out_v)

        pltpu.emit_pipeline(
            body,
            grid=(nidx // GW,),
            in_specs=[pl.BlockSpec((1, GW), index_map=lambda i: (0, i))],
            out_specs=[pl.BlockSpec((GW, EMB_DIM), index_map=lambda i: (i, 0))],
            core_axis_name=("core", "subcore"),
            dimension_semantics=(pltpu.PARALLEL,),
        )(idx_hbm, out_hbm)

    return gather_kernel(tables_lin, flat_idx)


def _mlp_body(*refs):
    n = len(CHUNKS)
    xn, ems = refs[0], refs[1:1 + n]
    w1n, w1s = refs[1 + n], refs[2 + n:2 + 2 * n]
    b1r, g1r, be1r, w2, b2r, g2r, be2r, w3, b3r, out = refs[2 + 2 * n:]
    s = np.float32(1.0 / np.sqrt(1.0 + EPS))
    h = jnp.dot(xn[...], w1n[...], preferred_element_type=jnp.float32)
    for em, w1e in zip(ems, w1s):
        h = h + jnp.dot(em[...], w1e[...], preferred_element_type=jnp.float32)
    a1 = g1r[...] * s
    h = h * a1 + (b1r[...] * a1 + be1r[...])
    h = jnp.maximum(h, 0.0)
    h2 = jnp.dot(h, w2[...], preferred_element_type=jnp.float32)
    a2 = g2r[...] * s
    h2 = h2 * a2 + (b2r[...] * a2 + be2r[...])
    h2 = jnp.maximum(h2, 0.0)
    out[...] = jnp.dot(h2, w3[...], preferred_element_type=jnp.float32) + b3r[...]


def _mlp(x_num, embs, W1n, W1s, b1, g1, be1, W2, b2, g2, be2, W3, b3):
    grid = (B // BLK,)
    row_spec = lambda w: pl.BlockSpec((BLK, w), lambda i: (i, 0))
    full_spec = lambda a: pl.BlockSpec(a.shape, lambda i: (0, 0))
    args = (x_num, *embs, W1n, *W1s, b1, g1, be1, W2, b2, g2, be2, W3, b3)
    in_specs = [row_spec(NUM_NUMERIC)] + [
        row_spec(nf * EMB_DIM) for nf in CHUNK_NF
    ] + [full_spec(a) for a in args[1 + len(CHUNK_NF):]]
    return pl.pallas_call(
        _mlp_body,
        grid=grid,
        in_specs=in_specs,
        out_specs=pl.BlockSpec((BLK, 1), lambda i: (i, 0)),
        out_shape=jax.ShapeDtypeStruct((B, 1), jnp.float32),
        compiler_params=pltpu.CompilerParams(dimension_semantics=("parallel",)),
    )(*args)


def _chunk_idx(x_cat_chunk, nf):
    f = jnp.arange(nf, dtype=jnp.int32)
    offs = (4 * PLANE * (f // FG) + f % FG)[None, :]
    return (x_cat_chunk * 4 + offs).reshape(1, B * nf)


def kernel(x_num, x_cat, emb_tables, W1, b1, g1, be1, W2, b2, g2, be2, W3, b3):
    tab_T = jnp.swapaxes(emb_tables, 1, 2)
    embs, W1s = [], []
    f0 = 0
    for (g0, ng), nf in zip(CHUNKS, CHUNK_NF):
        lin = _detranspose(tab_T, g0, ng).reshape(ng * PLANE * FG, EMB_DIM)
        idx = _chunk_idx(x_cat[:, f0:f0 + nf], nf)
        embs.append(_sc_gather(lin, idx, B * nf).reshape(B, nf * EMB_DIM))
        W1s.append(W1[NUM_NUMERIC + f0 * EMB_DIM:NUM_NUMERIC + (f0 + nf) * EMB_DIM])
        f0 += nf
    W1n = W1[:NUM_NUMERIC]
    vec = lambda v: v.reshape(1, -1)
    return _mlp(x_num, embs, W1n, W1s, vec(b1), vec(g1), vec(be1),
                W2, vec(b2), vec(g2), vec(be2), W3, vec(b3))
